# baseline (speedup 1.0000x reference)
import jax
import jax.numpy as jnp
from jax.experimental import pallas as pl
from jax.experimental.pallas import tpu as pltpu

VOCAB_ = 1000000
DIMS_ = 64
FOLD = 8
ROWS = VOCAB_ // FOLD
LANES = FOLD * DIMS_
BLK8 = 8192
NB = (ROWS + BLK8 - 1) // BLK8
NEG_INF = float("-inf")


def _body(idx_ref, table_ref, w_ref, b_ref, out_ref, z_ref, m_ref, s_ref):
    t = pl.program_id(0)

    @pl.when(t == 0)
    def _init():
        m_ref[...] = jnp.full_like(m_ref, NEG_INF)
        s_ref[...] = jnp.zeros_like(s_ref)

    @pl.when(t < NB)
    def _compute():
        i = idx_ref[0]
        p_sub = (i // FOLD) % 8
        q = i % FOLD
        row = table_ref[pl.ds(p_sub, 1), :]
        e = jnp.zeros((1, DIMS_), jnp.float32)
        for qq in range(FOLD):
            e = e + jnp.where(q == qq, 1.0, 0.0) * row[:, qq * DIMS_:(qq + 1) * DIMS_]
        e_tile = jnp.broadcast_to(
            jnp.concatenate([e] * FOLD, axis=1), (FOLD, LANES))
        lane = jax.lax.broadcasted_iota(jnp.int32, (FOLD, LANES), 1)
        sub = jax.lax.broadcasted_iota(jnp.int32, (FOLD, LANES), 0)
        e8 = jnp.where(lane // DIMS_ == sub, e_tile, 0.0)
        z = jax.lax.dot_general(
            e8, w_ref[...], (((1,), (1,)), ((), ())),
            preferred_element_type=jnp.float32)
        z = z + b_ref[...]
        z_ref[:, pl.ds(t * BLK8, BLK8)] = z
        k = jax.lax.broadcasted_iota(jnp.int32, (FOLD, BLK8), 1)
        p = jax.lax.broadcasted_iota(jnp.int32, (FOLD, BLK8), 0)
        gidx = FOLD * (t * BLK8 + k) + p
        zm = jnp.where(gidx < VOCAB_, z, NEG_INF)
        bm = jnp.max(zm, keepdims=True)
        new_m = jnp.maximum(m_ref[...], bm)
        s_ref[...] = s_ref[...] * jnp.exp(m_ref[...] - new_m) + jnp.sum(
            jnp.exp(zm - new_m), keepdims=True)
        m_ref[...] = new_m

    @pl.when(t >= NB)
    def _write():
        j = t - NB
        lse = m_ref[...] + jnp.log(s_ref[...])
        out_ref[...] = z_ref[:, pl.ds(j * BLK8, BLK8)] - lse


@jax.jit
def _run(inputs, tableF, WF, b8):
    grid_spec = pltpu.PrefetchScalarGridSpec(
        num_scalar_prefetch=1,
        grid=(2 * NB,),
        in_specs=[
            pl.BlockSpec((8, LANES), lambda t, idx: (idx[0] // (8 * FOLD), 0)),
            pl.BlockSpec((BLK8, LANES),
                         lambda t, idx: (jnp.minimum(t, NB - 1), 0)),
            pl.BlockSpec((FOLD, BLK8),
                         lambda t, idx: (0, jnp.minimum(t, NB - 1))),
        ],
        out_specs=pl.BlockSpec(
            (FOLD, BLK8), lambda t, idx: (0, jnp.where(t < NB, 0, t - NB))),
        scratch_shapes=[
            pltpu.VMEM((FOLD, NB * BLK8), jnp.float32),
            pltpu.VMEM((1, 1), jnp.float32),
            pltpu.VMEM((1, 1), jnp.float32),
        ],
    )
    return pl.pallas_call(
        _body,
        grid_spec=grid_spec,
        out_shape=jax.ShapeDtypeStruct((FOLD, ROWS), jnp.float32),
        compiler_params=pltpu.CompilerParams(
            dimension_semantics=("arbitrary",),
        ),
    )(inputs, tableF, WF, b8)


def kernel(inputs, table, W, b):
    idx = inputs.astype(jnp.int32)
    s = 1.0 + 0.0 * W[0, 0]
    WF = W.reshape(ROWS, LANES) * s
    tableF = table.reshape(ROWS, LANES) * s
    b8 = b.reshape(ROWS, FOLD).T
    out8 = _run(idx, tableF, WF, b8)
    return out8.T.reshape(1, VOCAB_)


# speedup vs baseline: 19.8450x; 19.8450x over previous
import jax
import jax.numpy as jnp
from jax.experimental import pallas as pl
from jax.experimental.pallas import tpu as pltpu

VOCAB_ = 1000000
DIMS_ = 64
BLK = 65536
NB = (VOCAB_ + BLK - 1) // BLK
NEG_INF = float("-inf")


def _body(idx_ref, table_ref, w_ref, b_ref, out_ref, z_ref, m_ref, s_ref):
    t = pl.program_id(0)

    @pl.when(t == 0)
    def _init():
        m_ref[...] = jnp.full_like(m_ref, NEG_INF)
        s_ref[...] = jnp.zeros_like(s_ref)

    @pl.when(t < NB)
    def _compute():
        lane = idx_ref[0] % 128
        tb = table_ref[...]
        li = jax.lax.broadcasted_iota(jnp.int32, (DIMS_, 128), 1)
        e_col = jnp.sum(jnp.where(li == lane, tb, 0.0), axis=1,
                        keepdims=True)
        z = jax.lax.dot_general(
            e_col, w_ref[...], (((0,), (0,)), ((), ())),
            preferred_element_type=jnp.float32)
        z = z + b_ref[...]
        z_ref[:, pl.ds(t * BLK, BLK)] = z
        col = t * BLK + jax.lax.broadcasted_iota(jnp.int32, (1, BLK), 1)
        zm = jnp.where(col < VOCAB_, z, NEG_INF)
        bm = jnp.max(zm, axis=1, keepdims=True)
        new_m = jnp.maximum(m_ref[...], bm)
        s_ref[...] = s_ref[...] * jnp.exp(m_ref[...] - new_m) + jnp.sum(
            jnp.exp(zm - new_m), axis=1, keepdims=True)
        m_ref[...] = new_m

    @pl.when(t >= NB)
    def _write():
        j = t - NB
        lse = m_ref[...] + jnp.log(s_ref[...])
        out_ref[...] = z_ref[:, pl.ds(j * BLK, BLK)] - lse


@jax.jit
def _run(inputs, tableT, WT, b2):
    grid_spec = pltpu.PrefetchScalarGridSpec(
        num_scalar_prefetch=1,
        grid=(2 * NB,),
        in_specs=[
            pl.BlockSpec((DIMS_, 128), lambda t, idx: (0, idx[0] // 128)),
            pl.BlockSpec((DIMS_, BLK), lambda t, idx: (0, jnp.minimum(t, NB - 1))),
            pl.BlockSpec((1, BLK), lambda t, idx: (0, jnp.minimum(t, NB - 1))),
        ],
        out_specs=pl.BlockSpec(
            (1, BLK), lambda t, idx: (0, jnp.where(t < NB, 0, t - NB))),
        scratch_shapes=[
            pltpu.VMEM((1, NB * BLK), jnp.float32),
            pltpu.VMEM((1, 1), jnp.float32),
            pltpu.VMEM((1, 1), jnp.float32),
        ],
    )
    return pl.pallas_call(
        _body,
        grid_spec=grid_spec,
        out_shape=jax.ShapeDtypeStruct((1, VOCAB_), jnp.float32),
        compiler_params=pltpu.CompilerParams(
            dimension_semantics=("arbitrary",),
        ),
    )(inputs, tableT, WT, b2)


def kernel(inputs, table, W, b):
    idx = inputs.astype(jnp.int32)
    return _run(idx, table.T, W.T, b.reshape(1, VOCAB_))


# speedup vs baseline: 20.5769x; 1.0369x over previous
import jax
import jax.numpy as jnp
from jax.experimental import pallas as pl
from jax.experimental.pallas import tpu as pltpu

VOCAB_ = 1000000
DIMS_ = 64
BLK = 65536
NB = (VOCAB_ + BLK - 1) // BLK
OBLK = 4 * BLK
NOB = (VOCAB_ + OBLK - 1) // OBLK
NEG_INF = float("-inf")


def _body(idx_ref, table_ref, w_ref, b_ref, out_ref, z_ref, m_ref, s_ref):
    t = pl.program_id(0)

    @pl.when(t == 0)
    def _init():
        m_ref[...] = jnp.full_like(m_ref, NEG_INF)
        s_ref[...] = jnp.zeros_like(s_ref)

    @pl.when(t < NB)
    def _compute():
        lane = idx_ref[0] % 128
        tb = table_ref[...]
        li = jax.lax.broadcasted_iota(jnp.int32, (DIMS_, 128), 1)
        e_col = jnp.sum(jnp.where(li == lane, tb, 0.0), axis=1,
                        keepdims=True)
        z = jax.lax.dot_general(
            e_col, w_ref[...], (((0,), (0,)), ((), ())),
            preferred_element_type=jnp.float32)
        z = z + b_ref[...]
        z_ref[:, pl.ds(t * BLK, BLK)] = z
        col = t * BLK + jax.lax.broadcasted_iota(jnp.int32, (1, BLK), 1)
        zm = jnp.where(col < VOCAB_, z, NEG_INF)
        bm = jnp.max(zm, axis=1, keepdims=True)
        new_m = jnp.maximum(m_ref[...], bm)
        s_ref[...] = s_ref[...] * jnp.exp(m_ref[...] - new_m) + jnp.sum(
            jnp.exp(zm - new_m), axis=1, keepdims=True)
        m_ref[...] = new_m

    @pl.when(t >= NB)
    def _write():
        j = t - NB
        lse = m_ref[...] + jnp.log(s_ref[...])
        out_ref[...] = z_ref[:, pl.ds(j * OBLK, OBLK)] - lse


@jax.jit
def _run(inputs, tableT, WT, b2):
    grid_spec = pltpu.PrefetchScalarGridSpec(
        num_scalar_prefetch=1,
        grid=(NB + NOB,),
        in_specs=[
            pl.BlockSpec((DIMS_, 128), lambda t, idx: (0, idx[0] // 128)),
            pl.BlockSpec((DIMS_, BLK), lambda t, idx: (0, jnp.minimum(t, NB - 1))),
            pl.BlockSpec((1, BLK), lambda t, idx: (0, jnp.minimum(t, NB - 1))),
        ],
        out_specs=pl.BlockSpec(
            (1, OBLK), lambda t, idx: (0, jnp.where(t < NB, 0, t - NB))),
        scratch_shapes=[
            pltpu.VMEM((1, NB * BLK), jnp.float32),
            pltpu.VMEM((1, 1), jnp.float32),
            pltpu.VMEM((1, 1), jnp.float32),
        ],
    )
    return pl.pallas_call(
        _body,
        grid_spec=grid_spec,
        out_shape=jax.ShapeDtypeStruct((1, VOCAB_), jnp.float32),
        compiler_params=pltpu.CompilerParams(
            dimension_semantics=("arbitrary",),
        ),
    )(inputs, tableT, WT, b2)


def kernel(inputs, table, W, b):
    idx = inputs.astype(jnp.int32)
    return _run(idx, table.T, W.T, b.reshape(1, VOCAB_))
